# baseline (speedup 1.0000x reference)
import functools

import jax
import jax.numpy as jnp
from jax import lax
from jax.experimental import pallas as pl
from jax.experimental.pallas import tpu as pltpu

HIDDEN = 2048
FF = 2048
E = 8
BT = 1024
FB = 256
HB = 256
NF = FF // FB


def _mm(a, b, dims):
    return lax.dot_general(a, b, (dims, ((), ())),
                           precision=lax.Precision.DEFAULT,
                           preferred_element_type=jnp.float32)


def _moe_body(x_ref, gw_ref, wg_ref, wu_ref, wd_ref,
              out_ref, logits_ref, probs_ref, inter_ref):
    e = pl.program_id(1)
    k = pl.program_id(2)
    par = lax.rem(e, 2)

    def _gate_up():
        xb = x_ref[...]
        gate = _mm(wg_ref[0], xb, ((1,), (0,)))
        up = _mm(wu_ref[0], xb, ((1,), (0,)))
        w_e = probs_ref[pl.ds(e, 1), :]
        inter = (jax.nn.silu(gate) * up * w_e).astype(jnp.bfloat16)
        inter_ref[par, pl.ds(k * FB, FB), :] = inter

    def _down_prev():
        inter_prev = inter_ref[1 - par]
        part = _mm(inter_prev, wd_ref[0], ((0,), (1,)))
        cur = out_ref[:, pl.ds(k * HB, HB)]
        out_ref[:, pl.ds(k * HB, HB)] = cur + part

    @pl.when((e == 0) & (k == 0))
    def _router():
        logits = _mm(gw_ref[...], x_ref[...], ((1,), (0,)))
        logits_ref[...] = logits
        m = jnp.max(logits, axis=0, keepdims=True)
        p = jnp.exp(logits - m)
        probs_ref[...] = p / jnp.sum(p, axis=0, keepdims=True)
        out_ref[...] = jnp.zeros_like(out_ref)

    @pl.when(e == 0)
    def _first():
        _gate_up()

    @pl.when((e > 0) & (e < E))
    def _steady():
        _down_prev()
        _gate_up()

    @pl.when(e == E)
    def _flush():
        _down_prev()


@functools.partial(jax.jit, static_argnums=())
def kernel(hidden_states, gate_w, gate_proj_w, up_proj_w, down_proj_w):
    b, s, h = hidden_states.shape
    t = b * s
    x = hidden_states.reshape(t, h).T.astype(jnp.bfloat16)

    nt = t // BT
    grid = (nt, E + 1, NF)

    def _w_cur_map(ti, ei, ki):
        return jnp.minimum(ei, E - 1), ki, 0

    def _wd_prev_map(ti, ei, ki):
        return jnp.maximum(ei - 1, 0), ki, 0

    out, logits_t = pl.pallas_call(
        _moe_body,
        grid=grid,
        in_specs=[
            pl.BlockSpec((h, BT), lambda ti, ei, ki: (0, ti)),
            pl.BlockSpec((E, h), lambda ti, ei, ki: (0, 0)),
            pl.BlockSpec((1, FB, h), _w_cur_map),
            pl.BlockSpec((1, FB, h), _w_cur_map),
            pl.BlockSpec((1, HB, FF), _wd_prev_map),
        ],
        out_specs=[
            pl.BlockSpec((BT, h), lambda ti, ei, ki: (ti, 0)),
            pl.BlockSpec((E, BT), lambda ti, ei, ki: (0, ti)),
        ],
        out_shape=[
            jax.ShapeDtypeStruct((t, h), jnp.float32),
            jax.ShapeDtypeStruct((E, t), jnp.float32),
        ],
        scratch_shapes=[
            pltpu.VMEM((E, BT), jnp.float32),
            pltpu.VMEM((2, FF, BT), jnp.bfloat16),
        ],
        compiler_params=pltpu.CompilerParams(
            dimension_semantics=("parallel", "arbitrary", "arbitrary"),
        ),
    )(x, gate_w, gate_proj_w, up_proj_w, down_proj_w)

    final = out.reshape(b, s, h)
    return final, logits_t.T


# speedup vs baseline: 1.0624x; 1.0624x over previous
import functools

import jax
import jax.numpy as jnp
from jax import lax
from jax.experimental import pallas as pl
from jax.experimental.pallas import tpu as pltpu

HIDDEN = 2048
FF = 2048
E = 8
BT = 1024
FB = 512
NF = FF // FB
NC = E * NF


def _mm(a, b, dims):
    return lax.dot_general(a, b, (dims, ((), ())),
                           precision=lax.Precision.DEFAULT,
                           preferred_element_type=jnp.float32)


def _moe_body(x_ref, gw_ref, wg_ref, wu_ref, wd_ref,
              out_ref, logits_ref, probs_ref, ia_ref, ib_ref):
    c = pl.program_id(1)
    par = lax.rem(c, 2)

    def _gate_up(dst_ref):
        xb = x_ref[...]
        gate = _mm(wg_ref[0], xb, ((1,), (0,)))
        up = _mm(wu_ref[0], xb, ((1,), (0,)))
        w_e = probs_ref[pl.ds(c // NF, 1), :]
        dst_ref[...] = (jax.nn.silu(gate) * up * w_e).astype(jnp.bfloat16)

    def _down_prev(src_ref):
        out_ref[...] += _mm(src_ref[...], wd_ref[0], ((0,), (1,)))

    @pl.when(c == 0)
    def _first():
        logits = _mm(gw_ref[...], x_ref[...], ((1,), (0,)))
        logits_ref[...] = logits
        m = jnp.max(logits, axis=0, keepdims=True)
        p = jnp.exp(logits - m)
        probs_ref[...] = p / jnp.sum(p, axis=0, keepdims=True)
        out_ref[...] = jnp.zeros_like(out_ref)
        _gate_up(ia_ref)

    @pl.when((c > 0) & (c < NC) & (par == 1))
    def _steady_odd():
        _down_prev(ia_ref)
        _gate_up(ib_ref)

    @pl.when((c > 0) & (c < NC) & (par == 0))
    def _steady_even():
        _down_prev(ib_ref)
        _gate_up(ia_ref)

    @pl.when(c == NC)
    def _flush():
        _down_prev(ib_ref)


@functools.partial(jax.jit, static_argnums=())
def kernel(hidden_states, gate_w, gate_proj_w, up_proj_w, down_proj_w):
    b, s, h = hidden_states.shape
    t = b * s
    x = hidden_states.reshape(t, h).T.astype(jnp.bfloat16)

    nt = t // BT
    grid = (nt, NC + 1)

    def _w_cur_map(ti, ci):
        cc = jnp.minimum(ci, NC - 1)
        return cc // NF, cc % NF, 0

    def _wd_prev_map(ti, ci):
        cp = jnp.maximum(ci - 1, 0)
        return cp // NF, 0, cp % NF

    out, logits_t = pl.pallas_call(
        _moe_body,
        grid=grid,
        in_specs=[
            pl.BlockSpec((h, BT), lambda ti, ci: (0, ti)),
            pl.BlockSpec((E, h), lambda ti, ci: (0, 0)),
            pl.BlockSpec((1, FB, h), _w_cur_map),
            pl.BlockSpec((1, FB, h), _w_cur_map),
            pl.BlockSpec((1, h, FB), _wd_prev_map),
        ],
        out_specs=[
            pl.BlockSpec((BT, h), lambda ti, ci: (ti, 0)),
            pl.BlockSpec((E, BT), lambda ti, ci: (0, ti)),
        ],
        out_shape=[
            jax.ShapeDtypeStruct((t, h), jnp.float32),
            jax.ShapeDtypeStruct((E, t), jnp.float32),
        ],
        scratch_shapes=[
            pltpu.VMEM((E, BT), jnp.float32),
            pltpu.VMEM((FB, BT), jnp.bfloat16),
            pltpu.VMEM((FB, BT), jnp.bfloat16),
        ],
        compiler_params=pltpu.CompilerParams(
            dimension_semantics=("parallel", "arbitrary"),
        ),
    )(x, gate_w, gate_proj_w, up_proj_w, down_proj_w)

    final = out.reshape(b, s, h)
    return final, logits_t.T


# speedup vs baseline: 1.0762x; 1.0130x over previous
import functools

import jax
import jax.numpy as jnp
from jax import lax
from jax.experimental import pallas as pl
from jax.experimental.pallas import tpu as pltpu

HIDDEN = 2048
FF = 2048
E = 8
BT = 2048
FB = 256
NF = FF // FB
NC = E * NF


def _mm(a, b, dims):
    return lax.dot_general(a, b, (dims, ((), ())),
                           precision=lax.Precision.DEFAULT,
                           preferred_element_type=jnp.float32)


def _moe_body(x_ref, gw_ref, wg_ref, wu_ref, wd_ref,
              out_ref, logits_ref, probs_ref, ia_ref, ib_ref):
    c = pl.program_id(1)
    par = lax.rem(c, 2)

    def _gate_up(dst_ref):
        xb = x_ref[...]
        gate = _mm(wg_ref[0], xb, ((1,), (0,)))
        up = _mm(wu_ref[0], xb, ((1,), (0,)))
        w_e = probs_ref[pl.ds(c // NF, 1), :]
        dst_ref[...] = (jax.nn.silu(gate) * up * w_e).astype(jnp.bfloat16)

    def _down_prev(src_ref):
        out_ref[...] += _mm(src_ref[...], wd_ref[0], ((0,), (1,)))

    @pl.when(c == 0)
    def _first():
        logits = _mm(gw_ref[...], x_ref[...], ((1,), (0,)))
        logits_ref[...] = logits
        m = jnp.max(logits, axis=0, keepdims=True)
        p = jnp.exp(logits - m)
        probs_ref[...] = p / jnp.sum(p, axis=0, keepdims=True)
        out_ref[...] = jnp.zeros_like(out_ref)
        _gate_up(ia_ref)

    @pl.when((c > 0) & (c < NC) & (par == 1))
    def _steady_odd():
        _down_prev(ia_ref)
        _gate_up(ib_ref)

    @pl.when((c > 0) & (c < NC) & (par == 0))
    def _steady_even():
        _down_prev(ib_ref)
        _gate_up(ia_ref)

    @pl.when(c == NC)
    def _flush():
        _down_prev(ib_ref)


@functools.partial(jax.jit, static_argnums=())
def kernel(hidden_states, gate_w, gate_proj_w, up_proj_w, down_proj_w):
    b, s, h = hidden_states.shape
    t = b * s
    x = hidden_states.reshape(t, h).T.astype(jnp.bfloat16)

    nt = t // BT
    grid = (nt, NC + 1)

    def _w_cur_map(ti, ci):
        cc = jnp.minimum(ci, NC - 1)
        return cc // NF, cc % NF, 0

    def _wd_prev_map(ti, ci):
        cp = jnp.maximum(ci - 1, 0)
        return cp // NF, 0, cp % NF

    out, logits_t = pl.pallas_call(
        _moe_body,
        grid=grid,
        in_specs=[
            pl.BlockSpec((h, BT), lambda ti, ci: (0, ti)),
            pl.BlockSpec((E, h), lambda ti, ci: (0, 0)),
            pl.BlockSpec((1, FB, h), _w_cur_map),
            pl.BlockSpec((1, FB, h), _w_cur_map),
            pl.BlockSpec((1, h, FB), _wd_prev_map),
        ],
        out_specs=[
            pl.BlockSpec((BT, h), lambda ti, ci: (ti, 0)),
            pl.BlockSpec((E, BT), lambda ti, ci: (0, ti)),
        ],
        out_shape=[
            jax.ShapeDtypeStruct((t, h), jnp.float32),
            jax.ShapeDtypeStruct((E, t), jnp.float32),
        ],
        scratch_shapes=[
            pltpu.VMEM((E, BT), jnp.float32),
            pltpu.VMEM((FB, BT), jnp.bfloat16),
            pltpu.VMEM((FB, BT), jnp.bfloat16),
        ],
        compiler_params=pltpu.CompilerParams(
            dimension_semantics=("parallel", "arbitrary"),
        ),
    )(x, gate_w, gate_proj_w, up_proj_w, down_proj_w)

    final = out.reshape(b, s, h)
    return final, logits_t.T


# speedup vs baseline: 1.0954x; 1.0178x over previous
import functools

import jax
import jax.numpy as jnp
from jax import lax
from jax.experimental import pallas as pl
from jax.experimental.pallas import tpu as pltpu

HIDDEN = 2048
FF = 2048
E = 8
BT = 1024
FB = 512


def _mm(a, b, dims):
    return lax.dot_general(a, b, (dims, ((), ())),
                           precision=lax.Precision.DEFAULT,
                           preferred_element_type=jnp.float32)


def _moe_body(x_ref, gw_ref, wg_ref, wu_ref, wd_ref,
              out_ref, logits_ref, probs_ref):
    e = pl.program_id(1)
    f = pl.program_id(2)

    @pl.when((e == 0) & (f == 0))
    def _router():
        logits = _mm(gw_ref[...], x_ref[...], ((1,), (0,)))
        logits_ref[...] = logits
        m = jnp.max(logits, axis=0, keepdims=True)
        p = jnp.exp(logits - m)
        probs_ref[...] = p / jnp.sum(p, axis=0, keepdims=True)
        out_ref[...] = jnp.zeros_like(out_ref)

    xb = x_ref[...]
    gate = _mm(wg_ref[0], xb, ((1,), (0,)))
    up = _mm(wu_ref[0], xb, ((1,), (0,)))
    w_e = probs_ref[pl.ds(e, 1), :]
    inter = (jax.nn.silu(gate) * up * w_e).astype(jnp.bfloat16)
    out_ref[...] += _mm(inter, wd_ref[0], ((0,), (1,)))


@functools.partial(jax.jit, static_argnums=())
def kernel(hidden_states, gate_w, gate_proj_w, up_proj_w, down_proj_w):
    b, s, h = hidden_states.shape
    t = b * s
    x = hidden_states.reshape(t, h).T.astype(jnp.bfloat16)

    nt = t // BT
    nf = FF // FB
    grid = (nt, E, nf)

    out, logits_t = pl.pallas_call(
        _moe_body,
        grid=grid,
        in_specs=[
            pl.BlockSpec((h, BT), lambda ti, ei, fi: (0, ti)),
            pl.BlockSpec((E, h), lambda ti, ei, fi: (0, 0)),
            pl.BlockSpec((1, FB, h), lambda ti, ei, fi: (ei, fi, 0)),
            pl.BlockSpec((1, FB, h), lambda ti, ei, fi: (ei, fi, 0)),
            pl.BlockSpec((1, h, FB), lambda ti, ei, fi: (ei, 0, fi)),
        ],
        out_specs=[
            pl.BlockSpec((BT, h), lambda ti, ei, fi: (ti, 0)),
            pl.BlockSpec((E, BT), lambda ti, ei, fi: (0, ti)),
        ],
        out_shape=[
            jax.ShapeDtypeStruct((t, h), jnp.float32),
            jax.ShapeDtypeStruct((E, t), jnp.float32),
        ],
        scratch_shapes=[pltpu.VMEM((E, BT), jnp.float32)],
        compiler_params=pltpu.CompilerParams(
            dimension_semantics=("parallel", "arbitrary", "arbitrary"),
        ),
    )(x, gate_w, gate_proj_w, up_proj_w, down_proj_w)

    final = out.reshape(b, s, h)
    return final, logits_t.T
